# Initial kernel scaffold; baseline (speedup 1.0000x reference)
import jax
import jax.numpy as jnp
from jax.experimental import pallas as pl


def kernel(x, edge_index, W1, b1, W_mu, b_mu, W_lv, b_lv):
    raise NotImplementedError("write your pallas kernel here")


# speedup vs baseline: 18.3313x; 18.3313x over previous
import functools

import jax
import jax.numpy as jnp
from jax import lax
from jax.experimental import pallas as pl
from jax.experimental.pallas import tpu as pltpu
from jax.experimental.pallas import tpu_sc as plsc

N = 10000
E = 320000
D_IN = 128
HID = 256
LAT = 128
NC, NS = 2, 16
CH = 80
NCHUNK = E // CH
DEG_PAD = 10240
N_PAD = 10112
RPT = N_PAD // NS
BLK = 1000


def _sc_mesh():
    return plsc.VectorSubcoreMesh(core_axis_name="c", subcore_axis_name="s",
                                  num_cores=NC, num_subcores=NS)


def _deg_sc(col3, ones, zeros1):
    cpt = NCHUNK // (NC * NS)

    @functools.partial(
        pl.kernel,
        out_type=jax.ShapeDtypeStruct((NC * DEG_PAD,), jnp.float32),
        mesh=_sc_mesh(),
        scratch_types=[
            pltpu.VMEM((cpt, CH), jnp.int32),
            pltpu.VMEM((CH,), jnp.float32),
            pltpu.VMEM_SHARED((DEG_PAD,), jnp.float32),
        ],
    )
    def k(col_hbm, ones_hbm, zeros_hbm, out_hbm, coli, ones_v, degs):
        c = lax.axis_index("c")
        s = lax.axis_index("s")
        wid = c * NS + s
        pltpu.sync_copy(col_hbm.at[wid], coli)
        pltpu.sync_copy(ones_hbm, ones_v)
        pltpu.sync_copy(zeros_hbm, degs.at[pl.ds(s * (DEG_PAD // NS), DEG_PAD // NS)])
        plsc.subcore_barrier()

        def body(j, carry):
            pltpu.sync_copy(ones_v, degs.at[coli.at[j]], add=True)
            return carry

        lax.fori_loop(0, cpt, body, 0)
        plsc.subcore_barrier()
        pltpu.sync_copy(degs.at[pl.ds(s * (DEG_PAD // NS), DEG_PAD // NS)],
                        out_hbm.at[pl.ds(c * DEG_PAD + s * (DEG_PAD // NS),
                                         DEG_PAD // NS)])

    return k(col3, ones, zeros1)


_AGG = None


def _agg_kernel():
    global _AGG
    if _AGG is not None:
        return _AGG
    cpt = NCHUNK // (NC * NS)

    @functools.partial(
        pl.kernel,
        out_type=jax.ShapeDtypeStruct((NC, N_PAD, D_IN), jnp.float32),
        mesh=_sc_mesh(),
        scratch_types=[
            pltpu.VMEM((cpt, CH), jnp.int32),
            pltpu.VMEM((cpt, CH), jnp.int32),
            pltpu.VMEM((CH, D_IN), jnp.float32),
            pltpu.VMEM_SHARED((N_PAD, D_IN), jnp.float32),
            pltpu.SemaphoreType.DMA,
        ],
    )
    def k(row_hbm, col_hbm, tab_hbm, zeros_hbm, out_hbm, rowi, coli, rowsb, acc, sem):
        c = lax.axis_index("c")
        s = lax.axis_index("s")
        wid = c * NS + s
        pltpu.sync_copy(row_hbm.at[wid], rowi)
        pltpu.sync_copy(col_hbm.at[wid], coli)
        pltpu.sync_copy(zeros_hbm, acc.at[pl.ds(s * RPT, RPT)])
        plsc.subcore_barrier()

        def body(j, carry):
            pltpu.async_copy(tab_hbm.at[rowi.at[j]], rowsb, sem).wait()
            pltpu.sync_copy(rowsb, acc.at[coli.at[j]], add=True)
            return carry

        lax.fori_loop(0, cpt, body, 0)
        plsc.subcore_barrier()
        pltpu.sync_copy(acc.at[pl.ds(s * RPT, RPT)],
                        out_hbm.at[c, pl.ds(s * RPT, RPT)])

    _AGG = k
    return k


def _tc_scale(x, degT):
    def body(x_ref, d_ref, o_ref):
        dinv = lax.rsqrt(d_ref[...] + 1.0)
        o_ref[...] = x_ref[...] * dinv

    return pl.pallas_call(
        body,
        grid=(N // BLK,),
        in_specs=[pl.BlockSpec((BLK, D_IN), lambda i: (i, 0)),
                  pl.BlockSpec((BLK, 1), lambda i: (i, 0))],
        out_specs=pl.BlockSpec((BLK, D_IN), lambda i: (i, 0)),
        out_shape=jax.ShapeDtypeStruct((N, D_IN), jnp.float32),
    )(x, degT)


def _tc_layer1(p1, x, degT, W1, b1r):
    def body(p_ref, x_ref, d_ref, w_ref, b_ref, oa_ref, ob_ref):
        dinv = lax.rsqrt(d_ref[...] + 1.0)
        xp = x_ref[...] * dinv
        u = (p_ref[0] + p_ref[1] + xp) * dinv
        h = jnp.dot(u, w_ref[...], preferred_element_type=jnp.float32) + b_ref[...]
        h = jnp.where(h > 0, h, jnp.exp(jnp.minimum(h, 0.0)) - 1.0)
        hp = h * dinv
        oa_ref[...] = hp[:, :D_IN]
        ob_ref[...] = hp[:, D_IN:]

    return pl.pallas_call(
        body,
        grid=(N // BLK,),
        in_specs=[pl.BlockSpec((NC, BLK, D_IN), lambda i: (0, i, 0)),
                  pl.BlockSpec((BLK, D_IN), lambda i: (i, 0)),
                  pl.BlockSpec((BLK, 1), lambda i: (i, 0)),
                  pl.BlockSpec((D_IN, HID), lambda i: (0, 0)),
                  pl.BlockSpec((1, HID), lambda i: (0, 0))],
        out_specs=[pl.BlockSpec((BLK, D_IN), lambda i: (i, 0)),
                   pl.BlockSpec((BLK, D_IN), lambda i: (i, 0))],
        out_shape=[jax.ShapeDtypeStruct((N, D_IN), jnp.float32),
                   jax.ShapeDtypeStruct((N, D_IN), jnp.float32)],
    )(p1, x, degT, W1, b1r)


def _tc_latent(pa, pb, ha, hb, degT, Wm, bmr, Wl, blr):
    def body(pa_ref, pb_ref, ha_ref, hb_ref, d_ref, wm_ref, bm_ref, wl_ref, bl_ref,
             mu_ref, lv_ref):
        dinv = lax.rsqrt(d_ref[...] + 1.0)
        g0 = (pa_ref[0] + pa_ref[1] + ha_ref[...]) * dinv
        g1 = (pb_ref[0] + pb_ref[1] + hb_ref[...]) * dinv
        mu_ref[...] = (jnp.dot(g0, wm_ref[0], preferred_element_type=jnp.float32)
                       + jnp.dot(g1, wm_ref[1], preferred_element_type=jnp.float32)
                       + bm_ref[...])
        lv_ref[...] = (jnp.dot(g0, wl_ref[0], preferred_element_type=jnp.float32)
                       + jnp.dot(g1, wl_ref[1], preferred_element_type=jnp.float32)
                       + bl_ref[...])

    wspec = pl.BlockSpec((NC, D_IN, LAT), lambda i: (0, 0, 0))
    bspec = pl.BlockSpec((1, LAT), lambda i: (0, 0))
    rspec = pl.BlockSpec((BLK, D_IN), lambda i: (i, 0))
    return pl.pallas_call(
        body,
        grid=(N // BLK,),
        in_specs=[pl.BlockSpec((NC, BLK, D_IN), lambda i: (0, i, 0)),
                  pl.BlockSpec((NC, BLK, D_IN), lambda i: (0, i, 0)),
                  rspec, rspec,
                  pl.BlockSpec((BLK, 1), lambda i: (i, 0)),
                  wspec, bspec, wspec, bspec],
        out_specs=[pl.BlockSpec((BLK, LAT), lambda i: (i, 0)),
                   pl.BlockSpec((BLK, LAT), lambda i: (i, 0))],
        out_shape=[jax.ShapeDtypeStruct((N, LAT), jnp.float32),
                   jax.ShapeDtypeStruct((N, LAT), jnp.float32)],
    )(pa, pb, ha, hb, degT, Wm, bmr, Wl, blr)


def kernel(x, edge_index, W1, b1, W_mu, b_mu, W_lv, b_lv):
    ei = edge_index.astype(jnp.int32)
    cpt2 = NCHUNK // (NC * NS)
    row32 = ei[0].reshape(NC * NS, cpt2, CH)
    col32 = ei[1].reshape(NC * NS, cpt2, CH)
    ones = jnp.ones((CH,), jnp.float32)
    zeros1 = jnp.zeros((DEG_PAD // NS,), jnp.float32)
    zeros2 = jnp.zeros((RPT, D_IN), jnp.float32)

    degf = _deg_sc(col32, ones, zeros1)
    degp = degf.reshape(NC, DEG_PAD)
    degT = (degp[0, :N] + degp[1, :N]).reshape(N, 1)

    agg = _agg_kernel()
    xp = _tc_scale(x, degT)
    p1 = agg(row32, col32, xp, zeros2)
    ha, hb = _tc_layer1(p1, x, degT, W1, b1.reshape(1, HID))
    pa = agg(row32, col32, ha, zeros2)
    pb = agg(row32, col32, hb, zeros2)
    mu, lv = _tc_latent(pa, pb, ha, hb, degT,
                        W_mu.reshape(NC, D_IN, LAT), b_mu.reshape(1, LAT),
                        W_lv.reshape(NC, D_IN, LAT), b_lv.reshape(1, LAT))
    return (mu, lv)


# speedup vs baseline: 23.0436x; 1.2571x over previous
import functools

import jax
import jax.numpy as jnp
from jax import lax
from jax.experimental import pallas as pl
from jax.experimental.pallas import tpu as pltpu
from jax.experimental.pallas import tpu_sc as plsc

N = 10000
E = 320000
D_IN = 128
HID = 256
LAT = 128
NC, NS = 2, 16
CH = 80
NCHUNK = E // CH
ACH = 80
ACPT = E // (NC * NS) // ACH
SG = 5
NSG = ACPT // SG
NBUF = 3
DEG_PAD = 10240
N_PAD = 10112
RPT = N_PAD // NS
BLK = 1000


def _sc_mesh():
    return plsc.VectorSubcoreMesh(core_axis_name="c", subcore_axis_name="s",
                                  num_cores=NC, num_subcores=NS)


def _deg_sc(col3, ones, zeros1):
    cpt = NCHUNK // (NC * NS)

    @functools.partial(
        pl.kernel,
        out_type=jax.ShapeDtypeStruct((NC * DEG_PAD,), jnp.float32),
        mesh=_sc_mesh(),
        scratch_types=[
            pltpu.VMEM((cpt, CH), jnp.int32),
            pltpu.VMEM((CH,), jnp.float32),
            pltpu.VMEM_SHARED((DEG_PAD,), jnp.float32),
        ],
    )
    def k(col_hbm, ones_hbm, zeros_hbm, out_hbm, coli, ones_v, degs):
        c = lax.axis_index("c")
        s = lax.axis_index("s")
        wid = c * NS + s
        pltpu.sync_copy(col_hbm.at[wid], coli)
        pltpu.sync_copy(ones_hbm, ones_v)
        pltpu.sync_copy(zeros_hbm, degs.at[pl.ds(s * (DEG_PAD // NS), DEG_PAD // NS)])
        plsc.subcore_barrier()

        def body(j, carry):
            pltpu.sync_copy(ones_v, degs.at[coli.at[j]], add=True)
            return carry

        lax.fori_loop(0, cpt, body, 0)
        plsc.subcore_barrier()
        pltpu.sync_copy(degs.at[pl.ds(s * (DEG_PAD // NS), DEG_PAD // NS)],
                        out_hbm.at[pl.ds(c * DEG_PAD + s * (DEG_PAD // NS),
                                         DEG_PAD // NS)])

    return k(col3, ones, zeros1)


_AGG = None


def _agg_kernel():
    global _AGG
    if _AGG is not None:
        return _AGG

    @functools.partial(
        pl.kernel,
        out_type=jax.ShapeDtypeStruct((NC, N_PAD, D_IN), jnp.float32),
        mesh=_sc_mesh(),
        scratch_types=[
            pltpu.VMEM((2, SG, ACH), jnp.int32),
            pltpu.VMEM((2, SG, ACH), jnp.int32),
            pltpu.VMEM((NBUF, ACH, D_IN), jnp.float32),
            pltpu.VMEM_SHARED((N_PAD, D_IN), jnp.float32),
            pltpu.SemaphoreType.DMA,
            pltpu.SemaphoreType.DMA,
            pltpu.SemaphoreType.DMA,
        ],
    )
    def k(row_hbm, col_hbm, tab_hbm, zeros_hbm, out_hbm,
          rowi, coli, bufs, acc, gsem, ssem, isem):
        c = lax.axis_index("c")
        s = lax.axis_index("s")
        wid = c * NS + s

        def idx_descs(t, p):
            return (pltpu.make_async_copy(row_hbm.at[wid, t], rowi.at[p], isem),
                    pltpu.make_async_copy(col_hbm.at[wid, t], coli.at[p], isem))

        def gat(j):
            t = j // SG
            u = lax.rem(j, SG)
            p = lax.rem(t, 2)
            slot = lax.rem(j, NBUF)
            return pltpu.make_async_copy(tab_hbm.at[rowi.at[p, u]],
                                         bufs.at[slot], gsem)

        def sct(j):
            t = j // SG
            u = lax.rem(j, SG)
            p = lax.rem(t, 2)
            slot = lax.rem(j, NBUF)
            return pltpu.make_async_copy(bufs.at[slot],
                                         acc.at[coli.at[p, u]], ssem)

        for d in idx_descs(0, 0):
            d.start()
        pltpu.sync_copy(zeros_hbm, acc.at[pl.ds(s * RPT, RPT)])
        plsc.subcore_barrier()
        for d in idx_descs(0, 0):
            d.wait()
        gat(0).start()

        def body(j, carry):
            t = j // SG
            u = lax.rem(j, SG)

            @pl.when(j >= 2)
            def _():
                sct(j - 2).wait()

            gat(j).wait()

            @pl.when(jnp.logical_and(u == 1, t + 1 < NSG))
            def _():
                for d in idx_descs(t + 1, lax.rem(t + 1, 2)):
                    d.start()

            sct(j).start(add=True)

            @pl.when(jnp.logical_and(u == SG - 1, t + 1 < NSG))
            def _():
                for d in idx_descs(t + 1, lax.rem(t + 1, 2)):
                    d.wait()

            @pl.when(j < ACPT - 1)
            def _():
                gat(j + 1).start()

            return carry

        lax.fori_loop(0, ACPT, body, 0)
        sct(ACPT - 2).wait()
        sct(ACPT - 1).wait()
        plsc.subcore_barrier()
        pltpu.sync_copy(acc.at[pl.ds(s * RPT, RPT)],
                        out_hbm.at[c, pl.ds(s * RPT, RPT)])

    _AGG = k
    return k


def _tc_scale(x, degT):
    def body(x_ref, d_ref, o_ref):
        dinv = lax.rsqrt(d_ref[...] + 1.0)
        o_ref[...] = x_ref[...] * dinv

    return pl.pallas_call(
        body,
        grid=(N // BLK,),
        in_specs=[pl.BlockSpec((BLK, D_IN), lambda i: (i, 0)),
                  pl.BlockSpec((BLK, 1), lambda i: (i, 0))],
        out_specs=pl.BlockSpec((BLK, D_IN), lambda i: (i, 0)),
        out_shape=jax.ShapeDtypeStruct((N, D_IN), jnp.float32),
    )(x, degT)


def _tc_layer1(p1, x, degT, W1, b1r):
    def body(p_ref, x_ref, d_ref, w_ref, b_ref, oa_ref, ob_ref):
        dinv = lax.rsqrt(d_ref[...] + 1.0)
        xp = x_ref[...] * dinv
        u = (p_ref[0] + p_ref[1] + xp) * dinv
        h = jnp.dot(u, w_ref[...], preferred_element_type=jnp.float32) + b_ref[...]
        h = jnp.where(h > 0, h, jnp.exp(jnp.minimum(h, 0.0)) - 1.0)
        hp = h * dinv
        oa_ref[...] = hp[:, :D_IN]
        ob_ref[...] = hp[:, D_IN:]

    return pl.pallas_call(
        body,
        grid=(N // BLK,),
        in_specs=[pl.BlockSpec((NC, BLK, D_IN), lambda i: (0, i, 0)),
                  pl.BlockSpec((BLK, D_IN), lambda i: (i, 0)),
                  pl.BlockSpec((BLK, 1), lambda i: (i, 0)),
                  pl.BlockSpec((D_IN, HID), lambda i: (0, 0)),
                  pl.BlockSpec((1, HID), lambda i: (0, 0))],
        out_specs=[pl.BlockSpec((BLK, D_IN), lambda i: (i, 0)),
                   pl.BlockSpec((BLK, D_IN), lambda i: (i, 0))],
        out_shape=[jax.ShapeDtypeStruct((N, D_IN), jnp.float32),
                   jax.ShapeDtypeStruct((N, D_IN), jnp.float32)],
    )(p1, x, degT, W1, b1r)


def _tc_latent(pa, pb, ha, hb, degT, Wm, bmr, Wl, blr):
    def body(pa_ref, pb_ref, ha_ref, hb_ref, d_ref, wm_ref, bm_ref, wl_ref, bl_ref,
             mu_ref, lv_ref):
        dinv = lax.rsqrt(d_ref[...] + 1.0)
        g0 = (pa_ref[0] + pa_ref[1] + ha_ref[...]) * dinv
        g1 = (pb_ref[0] + pb_ref[1] + hb_ref[...]) * dinv
        mu_ref[...] = (jnp.dot(g0, wm_ref[0], preferred_element_type=jnp.float32)
                       + jnp.dot(g1, wm_ref[1], preferred_element_type=jnp.float32)
                       + bm_ref[...])
        lv_ref[...] = (jnp.dot(g0, wl_ref[0], preferred_element_type=jnp.float32)
                       + jnp.dot(g1, wl_ref[1], preferred_element_type=jnp.float32)
                       + bl_ref[...])

    wspec = pl.BlockSpec((NC, D_IN, LAT), lambda i: (0, 0, 0))
    bspec = pl.BlockSpec((1, LAT), lambda i: (0, 0))
    rspec = pl.BlockSpec((BLK, D_IN), lambda i: (i, 0))
    return pl.pallas_call(
        body,
        grid=(N // BLK,),
        in_specs=[pl.BlockSpec((NC, BLK, D_IN), lambda i: (0, i, 0)),
                  pl.BlockSpec((NC, BLK, D_IN), lambda i: (0, i, 0)),
                  rspec, rspec,
                  pl.BlockSpec((BLK, 1), lambda i: (i, 0)),
                  wspec, bspec, wspec, bspec],
        out_specs=[pl.BlockSpec((BLK, LAT), lambda i: (i, 0)),
                   pl.BlockSpec((BLK, LAT), lambda i: (i, 0))],
        out_shape=[jax.ShapeDtypeStruct((N, LAT), jnp.float32),
                   jax.ShapeDtypeStruct((N, LAT), jnp.float32)],
    )(pa, pb, ha, hb, degT, Wm, bmr, Wl, blr)


def kernel(x, edge_index, W1, b1, W_mu, b_mu, W_lv, b_lv):
    ei = edge_index.astype(jnp.int32)
    cptd = NCHUNK // (NC * NS)
    cold = ei[1].reshape(NC * NS, cptd, CH)
    row32 = ei[0].reshape(NC * NS, NSG, SG, ACH)
    col32 = ei[1].reshape(NC * NS, NSG, SG, ACH)
    ones = jnp.ones((CH,), jnp.float32)
    zeros1 = jnp.zeros((DEG_PAD // NS,), jnp.float32)
    zeros2 = jnp.zeros((RPT, D_IN), jnp.float32)

    degf = _deg_sc(cold, ones, zeros1)
    degp = degf.reshape(NC, DEG_PAD)
    degT = (degp[0, :N] + degp[1, :N]).reshape(N, 1)

    agg = _agg_kernel()
    xp = _tc_scale(x, degT)
    p1 = agg(row32, col32, xp, zeros2)
    ha, hb = _tc_layer1(p1, x, degT, W1, b1.reshape(1, HID))
    pa = agg(row32, col32, ha, zeros2)
    pb = agg(row32, col32, hb, zeros2)
    mu, lv = _tc_latent(pa, pb, ha, hb, degT,
                        W_mu.reshape(NC, D_IN, LAT), b_mu.reshape(1, LAT),
                        W_lv.reshape(NC, D_IN, LAT), b_lv.reshape(1, LAT))
    return (mu, lv)


# speedup vs baseline: 32.7965x; 1.4232x over previous
import functools

import jax
import jax.numpy as jnp
from jax import lax
from jax.experimental import pallas as pl
from jax.experimental.pallas import tpu as pltpu
from jax.experimental.pallas import tpu_sc as plsc

N = 10000
E = 320000
D_IN = 128
HID = 256
LAT = 128
NC, NS = 2, 16
CH = 80
NCHUNK = E // CH
ACH = 80
ACPT = E // (NC * NS) // ACH
SG = 5
NSG = ACPT // SG
NBUF = 3
DEG_PAD = 10240
N_PAD = 10112
RPT = N_PAD // NS
BLK = 1000


def _sc_mesh():
    return plsc.VectorSubcoreMesh(core_axis_name="c", subcore_axis_name="s",
                                  num_cores=NC, num_subcores=NS)


def _deg_sc(col3, ones, zeros1):
    cpt = NCHUNK // (NC * NS)

    @functools.partial(
        pl.kernel,
        out_type=jax.ShapeDtypeStruct((NC * DEG_PAD,), jnp.float32),
        mesh=_sc_mesh(),
        scratch_types=[
            pltpu.VMEM((cpt, CH), jnp.int32),
            pltpu.VMEM((CH,), jnp.float32),
            pltpu.VMEM_SHARED((DEG_PAD,), jnp.float32),
        ],
    )
    def k(col_hbm, ones_hbm, zeros_hbm, out_hbm, coli, ones_v, degs):
        c = lax.axis_index("c")
        s = lax.axis_index("s")
        wid = c * NS + s
        pltpu.sync_copy(col_hbm.at[wid], coli)
        pltpu.sync_copy(ones_hbm, ones_v)
        pltpu.sync_copy(zeros_hbm, degs.at[pl.ds(s * (DEG_PAD // NS), DEG_PAD // NS)])
        plsc.subcore_barrier()

        def body(j, carry):
            pltpu.sync_copy(ones_v, degs.at[coli.at[j]], add=True)
            return carry

        lax.fori_loop(0, cpt, body, 0)
        plsc.subcore_barrier()
        pltpu.sync_copy(degs.at[pl.ds(s * (DEG_PAD // NS), DEG_PAD // NS)],
                        out_hbm.at[pl.ds(c * DEG_PAD + s * (DEG_PAD // NS),
                                         DEG_PAD // NS)])

    return k(col3, ones, zeros1)


_AGG = None


def _agg_kernel():
    global _AGG
    if _AGG is not None:
        return _AGG

    @functools.partial(
        pl.kernel,
        out_type=jax.ShapeDtypeStruct((NC, N_PAD, D_IN), jnp.float32),
        mesh=_sc_mesh(),
        scratch_types=[
            pltpu.VMEM((2, SG, ACH), jnp.int32),
            pltpu.VMEM((2, SG, ACH), jnp.int32),
            pltpu.VMEM((NBUF, ACH, D_IN), jnp.float32),
            pltpu.VMEM_SHARED((N_PAD, D_IN), jnp.float32),
            pltpu.SemaphoreType.DMA,
            pltpu.SemaphoreType.DMA,
            pltpu.SemaphoreType.DMA,
        ],
    )
    def k(row_hbm, col_hbm, tab_hbm, zeros_hbm, out_hbm,
          rowi, coli, bufs, acc, gsem, ssem, isem):
        c = lax.axis_index("c")
        s = lax.axis_index("s")
        wid = c * NS + s

        def idx_descs(t, p):
            return (pltpu.make_async_copy(row_hbm.at[wid, t], rowi.at[p], isem),
                    pltpu.make_async_copy(col_hbm.at[wid, t], coli.at[p], isem))

        def gat(j):
            t = j // SG
            u = lax.rem(j, SG)
            p = lax.rem(t, 2)
            slot = lax.rem(j, NBUF)
            return pltpu.make_async_copy(tab_hbm.at[rowi.at[p, u]],
                                         bufs.at[slot], gsem)

        def sct(j):
            t = j // SG
            u = lax.rem(j, SG)
            p = lax.rem(t, 2)
            slot = lax.rem(j, NBUF)
            return pltpu.make_async_copy(bufs.at[slot],
                                         acc.at[coli.at[p, u]], ssem)

        for d in idx_descs(0, 0):
            d.start()
        pltpu.sync_copy(zeros_hbm, acc.at[pl.ds(s * RPT, RPT)])
        plsc.subcore_barrier()
        for d in idx_descs(0, 0):
            d.wait()
        gat(0).start()
        gat(1).start()

        def body(j, carry):
            t = j // SG
            u = lax.rem(j, SG)
            t2 = (j + 2) // SG

            gat(j).wait()

            @pl.when(jnp.logical_and(u == 1, t + 1 < NSG))
            def _():
                for d in idx_descs(t + 1, lax.rem(t + 1, 2)):
                    d.start()

            sct(j).start(add=True)

            @pl.when(jnp.logical_and(lax.rem(j + 2, SG) == 0, t2 < NSG))
            def _():
                for d in idx_descs(t2, lax.rem(t2, 2)):
                    d.wait()

            @pl.when(j + 2 < ACPT)
            def _():
                @pl.when(j >= 1)
                def _():
                    sct(j - 1).wait()
                gat(j + 2).start()

            return carry

        lax.fori_loop(0, ACPT, body, 0)
        sct(ACPT - 3).wait()
        sct(ACPT - 2).wait()
        sct(ACPT - 1).wait()
        plsc.subcore_barrier()
        pltpu.sync_copy(acc.at[pl.ds(s * RPT, RPT)],
                        out_hbm.at[c, pl.ds(s * RPT, RPT)])

    _AGG = k
    return k


def _tc_scale(x, degT):
    def body(x_ref, d_ref, o_ref):
        dinv = lax.rsqrt(d_ref[...] + 1.0)
        o_ref[...] = x_ref[...] * dinv

    return pl.pallas_call(
        body,
        grid=(N // BLK,),
        in_specs=[pl.BlockSpec((BLK, D_IN), lambda i: (i, 0)),
                  pl.BlockSpec((BLK, 1), lambda i: (i, 0))],
        out_specs=pl.BlockSpec((BLK, D_IN), lambda i: (i, 0)),
        out_shape=jax.ShapeDtypeStruct((N, D_IN), jnp.float32),
    )(x, degT)


def _tc_layer1(p1, x, degT, W1, b1r):
    def body(p_ref, x_ref, d_ref, w_ref, b_ref, oa_ref, ob_ref):
        dinv = lax.rsqrt(d_ref[...] + 1.0)
        xp = x_ref[...] * dinv
        u = (p_ref[0] + p_ref[1] + xp) * dinv
        h = jnp.dot(u, w_ref[...], preferred_element_type=jnp.float32) + b_ref[...]
        h = jnp.where(h > 0, h, jnp.exp(jnp.minimum(h, 0.0)) - 1.0)
        hp = h * dinv
        oa_ref[...] = hp[:, :D_IN]
        ob_ref[...] = hp[:, D_IN:]

    return pl.pallas_call(
        body,
        grid=(N // BLK,),
        in_specs=[pl.BlockSpec((NC, BLK, D_IN), lambda i: (0, i, 0)),
                  pl.BlockSpec((BLK, D_IN), lambda i: (i, 0)),
                  pl.BlockSpec((BLK, 1), lambda i: (i, 0)),
                  pl.BlockSpec((D_IN, HID), lambda i: (0, 0)),
                  pl.BlockSpec((1, HID), lambda i: (0, 0))],
        out_specs=[pl.BlockSpec((BLK, D_IN), lambda i: (i, 0)),
                   pl.BlockSpec((BLK, D_IN), lambda i: (i, 0))],
        out_shape=[jax.ShapeDtypeStruct((N, D_IN), jnp.float32),
                   jax.ShapeDtypeStruct((N, D_IN), jnp.float32)],
    )(p1, x, degT, W1, b1r)


def _tc_latent(pa, pb, ha, hb, degT, Wm, bmr, Wl, blr):
    def body(pa_ref, pb_ref, ha_ref, hb_ref, d_ref, wm_ref, bm_ref, wl_ref, bl_ref,
             mu_ref, lv_ref):
        dinv = lax.rsqrt(d_ref[...] + 1.0)
        g0 = (pa_ref[0] + pa_ref[1] + ha_ref[...]) * dinv
        g1 = (pb_ref[0] + pb_ref[1] + hb_ref[...]) * dinv
        mu_ref[...] = (jnp.dot(g0, wm_ref[0], preferred_element_type=jnp.float32)
                       + jnp.dot(g1, wm_ref[1], preferred_element_type=jnp.float32)
                       + bm_ref[...])
        lv_ref[...] = (jnp.dot(g0, wl_ref[0], preferred_element_type=jnp.float32)
                       + jnp.dot(g1, wl_ref[1], preferred_element_type=jnp.float32)
                       + bl_ref[...])

    wspec = pl.BlockSpec((NC, D_IN, LAT), lambda i: (0, 0, 0))
    bspec = pl.BlockSpec((1, LAT), lambda i: (0, 0))
    rspec = pl.BlockSpec((BLK, D_IN), lambda i: (i, 0))
    return pl.pallas_call(
        body,
        grid=(N // BLK,),
        in_specs=[pl.BlockSpec((NC, BLK, D_IN), lambda i: (0, i, 0)),
                  pl.BlockSpec((NC, BLK, D_IN), lambda i: (0, i, 0)),
                  rspec, rspec,
                  pl.BlockSpec((BLK, 1), lambda i: (i, 0)),
                  wspec, bspec, wspec, bspec],
        out_specs=[pl.BlockSpec((BLK, LAT), lambda i: (i, 0)),
                   pl.BlockSpec((BLK, LAT), lambda i: (i, 0))],
        out_shape=[jax.ShapeDtypeStruct((N, LAT), jnp.float32),
                   jax.ShapeDtypeStruct((N, LAT), jnp.float32)],
    )(pa, pb, ha, hb, degT, Wm, bmr, Wl, blr)


def kernel(x, edge_index, W1, b1, W_mu, b_mu, W_lv, b_lv):
    ei = edge_index.astype(jnp.int32)
    cptd = NCHUNK // (NC * NS)
    cold = ei[1].reshape(NC * NS, cptd, CH)
    row32 = ei[0].reshape(NC * NS, NSG, SG, ACH)
    col32 = ei[1].reshape(NC * NS, NSG, SG, ACH)
    ones = jnp.ones((CH,), jnp.float32)
    zeros1 = jnp.zeros((DEG_PAD // NS,), jnp.float32)
    zeros2 = jnp.zeros((RPT, D_IN), jnp.float32)

    degf = _deg_sc(cold, ones, zeros1)
    degp = degf.reshape(NC, DEG_PAD)
    degT = (degp[0, :N] + degp[1, :N]).reshape(N, 1)

    agg = _agg_kernel()
    xp = _tc_scale(x, degT)
    p1 = agg(row32, col32, xp, zeros2)
    ha, hb = _tc_layer1(p1, x, degT, W1, b1.reshape(1, HID))
    pa = agg(row32, col32, ha, zeros2)
    pb = agg(row32, col32, hb, zeros2)
    mu, lv = _tc_latent(pa, pb, ha, hb, degT,
                        W_mu.reshape(NC, D_IN, LAT), b_mu.reshape(1, LAT),
                        W_lv.reshape(NC, D_IN, LAT), b_lv.reshape(1, LAT))
    return (mu, lv)


# speedup vs baseline: 34.8654x; 1.0631x over previous
import functools

import jax
import jax.numpy as jnp
from jax import lax
from jax.experimental import pallas as pl
from jax.experimental.pallas import tpu as pltpu
from jax.experimental.pallas import tpu_sc as plsc

N = 10000
E = 320000
D_IN = 128
HID = 256
LAT = 128
NC, NS = 2, 16
CH = 80
NCHUNK = E // CH
ACH = 80
ACPT = E // (NC * NS) // ACH
SG = 5
NSG = ACPT // SG
NBUF = 4
DEG_PAD = 10240
N_PAD = 10112
RPT = N_PAD // NS
BLK = 1000


def _sc_mesh():
    return plsc.VectorSubcoreMesh(core_axis_name="c", subcore_axis_name="s",
                                  num_cores=NC, num_subcores=NS)


def _deg_sc(col4, ones, zeros1):
    STR = DEG_PAD // NS

    @functools.partial(
        pl.kernel,
        out_type=jax.ShapeDtypeStruct((NC * DEG_PAD,), jnp.float32),
        mesh=_sc_mesh(),
        scratch_types=[
            pltpu.VMEM((2, SG, ACH), jnp.int32),
            pltpu.VMEM((ACH,), jnp.float32),
            pltpu.VMEM_SHARED((DEG_PAD,), jnp.float32),
            pltpu.SemaphoreType.DMA,
            pltpu.SemaphoreType.DMA,
        ],
    )
    def k(col_hbm, ones_hbm, zeros_hbm, out_hbm, coli, ones_v, degs, ssem, isem):
        c = lax.axis_index("c")
        s = lax.axis_index("s")
        wid = c * NS + s

        def idx_d(t, p):
            return pltpu.make_async_copy(col_hbm.at[wid, t], coli.at[p], isem)

        def sct(j):
            t = j // SG
            u = lax.rem(j, SG)
            p = lax.rem(t, 2)
            return pltpu.make_async_copy(ones_v, degs.at[coli.at[p, u]], ssem)

        idx_d(0, 0).start()
        pltpu.sync_copy(ones_hbm, ones_v)
        pltpu.sync_copy(zeros_hbm, degs.at[pl.ds(s * STR, STR)])
        plsc.subcore_barrier()
        idx_d(0, 0).wait()

        def body(j, carry):
            t = j // SG
            u = lax.rem(j, SG)

            @pl.when(jnp.logical_and(u == 0, t >= 1))
            def _():
                for kk in range(SG):
                    sct(j - SG + kk).wait()
                idx_d(t, lax.rem(t, 2)).wait()

            @pl.when(jnp.logical_and(u == 0, t + 1 < NSG))
            def _():
                idx_d(t + 1, lax.rem(t + 1, 2)).start()

            sct(j).start(add=True)
            return carry

        lax.fori_loop(0, ACPT, body, 0)
        for kk in range(SG):
            sct(ACPT - SG + kk).wait()
        plsc.subcore_barrier()
        pltpu.sync_copy(degs.at[pl.ds(s * STR, STR)],
                        out_hbm.at[pl.ds(c * DEG_PAD + s * STR, STR)])

    return k(col4, ones, zeros1)


_AGG = None


def _agg_kernel():
    global _AGG
    if _AGG is not None:
        return _AGG

    @functools.partial(
        pl.kernel,
        out_type=jax.ShapeDtypeStruct((NC, N_PAD, D_IN), jnp.float32),
        mesh=_sc_mesh(),
        scratch_types=[
            pltpu.VMEM((2, SG, ACH), jnp.int32),
            pltpu.VMEM((2, SG, ACH), jnp.int32),
            pltpu.VMEM((NBUF, ACH, D_IN), jnp.float32),
            pltpu.VMEM_SHARED((N_PAD, D_IN), jnp.float32),
            pltpu.SemaphoreType.DMA,
            pltpu.SemaphoreType.DMA,
            pltpu.SemaphoreType.DMA,
        ],
    )
    def k(row_hbm, col_hbm, tab_hbm, zeros_hbm, out_hbm,
          rowi, coli, bufs, acc, gsem, ssem, isem):
        c = lax.axis_index("c")
        s = lax.axis_index("s")
        wid = c * NS + s

        def idx_descs(t, p):
            return (pltpu.make_async_copy(row_hbm.at[wid, t], rowi.at[p], isem),
                    pltpu.make_async_copy(col_hbm.at[wid, t], coli.at[p], isem))

        def gat(j):
            t = j // SG
            u = lax.rem(j, SG)
            p = lax.rem(t, 2)
            slot = lax.rem(j, NBUF)
            return pltpu.make_async_copy(tab_hbm.at[rowi.at[p, u]],
                                         bufs.at[slot], gsem)

        def sct(j):
            t = j // SG
            u = lax.rem(j, SG)
            p = lax.rem(t, 2)
            slot = lax.rem(j, NBUF)
            return pltpu.make_async_copy(bufs.at[slot],
                                         acc.at[coli.at[p, u]], ssem)

        for d in idx_descs(0, 0):
            d.start()
        pltpu.sync_copy(zeros_hbm, acc.at[pl.ds(s * RPT, RPT)])
        plsc.subcore_barrier()
        for d in idx_descs(0, 0):
            d.wait()
        gat(0).start()
        gat(1).start()
        gat(2).start()

        def body(j, carry):
            t = j // SG
            u = lax.rem(j, SG)
            t3 = (j + 3) // SG

            gat(j).wait()

            @pl.when(jnp.logical_and(u == 0, t + 1 < NSG))
            def _():
                for d in idx_descs(t + 1, lax.rem(t + 1, 2)):
                    d.start()

            sct(j).start(add=True)

            @pl.when(jnp.logical_and(lax.rem(j + 3, SG) == 0, t3 < NSG))
            def _():
                for d in idx_descs(t3, lax.rem(t3, 2)):
                    d.wait()

            @pl.when(j + 3 < ACPT)
            def _():
                @pl.when(j >= 1)
                def _():
                    sct(j - 1).wait()
                gat(j + 3).start()

            return carry

        lax.fori_loop(0, ACPT, body, 0)
        sct(ACPT - 4).wait()
        sct(ACPT - 3).wait()
        sct(ACPT - 2).wait()
        sct(ACPT - 1).wait()
        plsc.subcore_barrier()
        pltpu.sync_copy(acc.at[pl.ds(s * RPT, RPT)],
                        out_hbm.at[c, pl.ds(s * RPT, RPT)])

    _AGG = k
    return k


def _tc_scale(x, degT):
    def body(x_ref, d_ref, o_ref):
        dinv = lax.rsqrt(d_ref[...] + 1.0)
        o_ref[...] = x_ref[...] * dinv

    return pl.pallas_call(
        body,
        grid=(N // BLK,),
        in_specs=[pl.BlockSpec((BLK, D_IN), lambda i: (i, 0)),
                  pl.BlockSpec((BLK, 1), lambda i: (i, 0))],
        out_specs=pl.BlockSpec((BLK, D_IN), lambda i: (i, 0)),
        out_shape=jax.ShapeDtypeStruct((N, D_IN), jnp.float32),
    )(x, degT)


def _tc_layer1(p1, x, degT, W1, b1r):
    def body(p_ref, x_ref, d_ref, w_ref, b_ref, oa_ref, ob_ref):
        dinv = lax.rsqrt(d_ref[...] + 1.0)
        xp = x_ref[...] * dinv
        u = (p_ref[0] + p_ref[1] + xp) * dinv
        h = jnp.dot(u, w_ref[...], preferred_element_type=jnp.float32) + b_ref[...]
        h = jnp.where(h > 0, h, jnp.exp(jnp.minimum(h, 0.0)) - 1.0)
        hp = h * dinv
        oa_ref[...] = hp[:, :D_IN]
        ob_ref[...] = hp[:, D_IN:]

    return pl.pallas_call(
        body,
        grid=(N // BLK,),
        in_specs=[pl.BlockSpec((NC, BLK, D_IN), lambda i: (0, i, 0)),
                  pl.BlockSpec((BLK, D_IN), lambda i: (i, 0)),
                  pl.BlockSpec((BLK, 1), lambda i: (i, 0)),
                  pl.BlockSpec((D_IN, HID), lambda i: (0, 0)),
                  pl.BlockSpec((1, HID), lambda i: (0, 0))],
        out_specs=[pl.BlockSpec((BLK, D_IN), lambda i: (i, 0)),
                   pl.BlockSpec((BLK, D_IN), lambda i: (i, 0))],
        out_shape=[jax.ShapeDtypeStruct((N, D_IN), jnp.float32),
                   jax.ShapeDtypeStruct((N, D_IN), jnp.float32)],
    )(p1, x, degT, W1, b1r)


def _tc_latent(pa, pb, ha, hb, degT, Wm, bmr, Wl, blr):
    def body(pa_ref, pb_ref, ha_ref, hb_ref, d_ref, wm_ref, bm_ref, wl_ref, bl_ref,
             mu_ref, lv_ref):
        dinv = lax.rsqrt(d_ref[...] + 1.0)
        g0 = (pa_ref[0] + pa_ref[1] + ha_ref[...]) * dinv
        g1 = (pb_ref[0] + pb_ref[1] + hb_ref[...]) * dinv
        mu_ref[...] = (jnp.dot(g0, wm_ref[0], preferred_element_type=jnp.float32)
                       + jnp.dot(g1, wm_ref[1], preferred_element_type=jnp.float32)
                       + bm_ref[...])
        lv_ref[...] = (jnp.dot(g0, wl_ref[0], preferred_element_type=jnp.float32)
                       + jnp.dot(g1, wl_ref[1], preferred_element_type=jnp.float32)
                       + bl_ref[...])

    wspec = pl.BlockSpec((NC, D_IN, LAT), lambda i: (0, 0, 0))
    bspec = pl.BlockSpec((1, LAT), lambda i: (0, 0))
    rspec = pl.BlockSpec((BLK, D_IN), lambda i: (i, 0))
    return pl.pallas_call(
        body,
        grid=(N // BLK,),
        in_specs=[pl.BlockSpec((NC, BLK, D_IN), lambda i: (0, i, 0)),
                  pl.BlockSpec((NC, BLK, D_IN), lambda i: (0, i, 0)),
                  rspec, rspec,
                  pl.BlockSpec((BLK, 1), lambda i: (i, 0)),
                  wspec, bspec, wspec, bspec],
        out_specs=[pl.BlockSpec((BLK, LAT), lambda i: (i, 0)),
                   pl.BlockSpec((BLK, LAT), lambda i: (i, 0))],
        out_shape=[jax.ShapeDtypeStruct((N, LAT), jnp.float32),
                   jax.ShapeDtypeStruct((N, LAT), jnp.float32)],
    )(pa, pb, ha, hb, degT, Wm, bmr, Wl, blr)


def kernel(x, edge_index, W1, b1, W_mu, b_mu, W_lv, b_lv):
    ei = edge_index.astype(jnp.int32)
    row32 = ei[0].reshape(NC * NS, NSG, SG, ACH)
    col32 = ei[1].reshape(NC * NS, NSG, SG, ACH)
    ones = jnp.ones((ACH,), jnp.float32)
    zeros1 = jnp.zeros((DEG_PAD // NS,), jnp.float32)
    zeros2 = jnp.zeros((RPT, D_IN), jnp.float32)

    degf = _deg_sc(col32, ones, zeros1)
    degp = degf.reshape(NC, DEG_PAD)
    degT = (degp[0, :N] + degp[1, :N]).reshape(N, 1)

    agg = _agg_kernel()
    xp = _tc_scale(x, degT)
    p1 = agg(row32, col32, xp, zeros2)
    ha, hb = _tc_layer1(p1, x, degT, W1, b1.reshape(1, HID))
    pa = agg(row32, col32, ha, zeros2)
    pb = agg(row32, col32, hb, zeros2)
    mu, lv = _tc_latent(pa, pb, ha, hb, degT,
                        W_mu.reshape(NC, D_IN, LAT), b_mu.reshape(1, LAT),
                        W_lv.reshape(NC, D_IN, LAT), b_lv.reshape(1, LAT))
    return (mu, lv)
